# initial kernel scaffold (unmeasured)
import jax
import jax.numpy as jnp
from jax import lax
from jax.experimental import pallas as pl
from jax.experimental.pallas import tpu as pltpu

N_DEV = 8
B = 2
SQ = 512
F = 768
H_LOC = 8
DH = 64
SKV_LOC = 512
QB = 64
R = 4


def kernel(x, Wq, K_ext, V_ext, Wo):
    kv = jnp.stack(
        [K_ext.astype(jnp.bfloat16), V_ext.astype(jnp.bfloat16)], axis=0
    )

    def body(x_ref, wq_ref, kv_ref, wo_ref, out_ref,
             kvfull, rsbuf, pscratch,
             kv_s, kv_r, rs_s, rs_r, ag_s, ag_r, loc_sem):
        me = lax.axis_index("i")

        kv_sends = []
        for o in range(1, N_DEV):
            d = (me + o) % N_DEV
            rdma = pltpu.make_async_remote_copy(
                src_ref=kv_ref.at[:, :, :, pl.ds(H_LOC * d, H_LOC), :],
                dst_ref=kvfull.at[me],
                send_sem=kv_s.at[o - 1],
                recv_sem=kv_r.at[o - 1],
                device_id=(d,),
            )
            rdma.start()
            kv_sends.append(rdma)

        own = pltpu.make_async_copy(
            kv_ref.at[:, :, :, pl.ds(H_LOC * me, H_LOC), :],
            kvfull.at[me],
            loc_sem,
        )
        own.start()

        xv = x_ref[...].reshape(B * SQ, F)
        q2d = lax.dot_general(
            xv, wq_ref[...], (((1,), (0,)), ((), ())),
            preferred_element_type=jnp.float32,
        )
        qb16 = q2d.astype(jnp.bfloat16)

        own.wait()
        for o in range(1, N_DEV):
            s = (me + (N_DEV - o)) % N_DEV
            pltpu.make_async_remote_copy(
                src_ref=kv_ref.at[:, :, :, pl.ds(0, H_LOC), :],
                dst_ref=kvfull.at[s],
                send_sem=kv_s.at[o - 1],
                recv_sem=kv_r.at[o - 1],
                device_id=(me,),
            ).wait_recv()

        for b in range(B):
            kall = kvfull[:, 0, b].reshape(N_DEV, 2, R, QB, H_LOC, DH)
            vall = kvfull[:, 1, b].reshape(N_DEV, 2, R, QB, H_LOC, DH)
            ctx_halves = []
            for c in range(R):
                kc = kall[:, :, c].reshape(N_DEV * 2 * QB, H_LOC, DH)
                vc = vall[:, :, c].reshape(N_DEV * 2 * QB, H_LOC, DH)
                qrows = jnp.concatenate(
                    [
                        qb16[SQ * b + QB * c: SQ * b + QB * (c + 1)],
                        qb16[SQ * b + QB * (c + R): SQ * b + QB * (c + R + 1)],
                    ],
                    axis=0,
                ).reshape(2 * QB, H_LOC, DH)
                scores = lax.dot_general(
                    qrows, kc, (((2,), (2,)), ((1,), (1,))),
                    preferred_element_type=jnp.float32,
                )
                scores = scores * 0.125
                m = jnp.max(scores, axis=-1, keepdims=True)
                w = jnp.exp(scores - m)
                w = w / jnp.sum(w, axis=-1, keepdims=True)
                ctx = lax.dot_general(
                    w.astype(jnp.bfloat16), vc,
                    (((2,), (0,)), ((0,), (1,))),
                    preferred_element_type=jnp.float32,
                )
                ctx_halves.append(ctx.transpose(1, 0, 2).reshape(2 * QB, H_LOC * DH))
            blocks = []
            for qb in range(SQ // QB):
                half = 0 if qb < R else 1
                blocks.append(ctx_halves[qb % R][QB * half: QB * (half + 1)])
            ctx_b = jnp.concatenate(blocks, axis=0)
            pscratch[b] = lax.dot_general(
                ctx_b, wo_ref[...], (((1,), (0,)), ((), ())),
                preferred_element_type=jnp.float32,
            )

        rs_own = pltpu.make_async_copy(
            pscratch.at[:, pl.ds(QB * me, QB), :], rsbuf.at[me], loc_sem
        )
        rs_own.start()
        rs_sends = []
        for o in range(1, N_DEV):
            d = (me + o) % N_DEV
            rdma = pltpu.make_async_remote_copy(
                src_ref=pscratch.at[:, pl.ds(QB * d, QB), :],
                dst_ref=rsbuf.at[me],
                send_sem=rs_s.at[o - 1],
                recv_sem=rs_r.at[o - 1],
                device_id=(d,),
            )
            rdma.start()
            rs_sends.append(rdma)
        rs_own.wait()
        for o in range(1, N_DEV):
            s = (me + (N_DEV - o)) % N_DEV
            pltpu.make_async_remote_copy(
                src_ref=pscratch.at[:, pl.ds(0, QB), :],
                dst_ref=rsbuf.at[s],
                send_sem=rs_s.at[o - 1],
                recv_sem=rs_r.at[o - 1],
                device_id=(me,),
            ).wait_recv()
        red = jnp.sum(rsbuf[...], axis=0)
        out_ref[:, pl.ds(QB * me, QB), :] = red

        ag_sends = []
        for o in range(1, N_DEV):
            d = (me + o) % N_DEV
            rdma = pltpu.make_async_remote_copy(
                src_ref=out_ref.at[:, pl.ds(QB * me, QB), :],
                dst_ref=out_ref.at[:, pl.ds(QB * me, QB), :],
                send_sem=ag_s.at[o - 1],
                recv_sem=ag_r.at[o - 1],
                device_id=(d,),
            )
            rdma.start()
            ag_sends.append(rdma)
        for o in range(1, N_DEV):
            s = (me + (N_DEV - o)) % N_DEV
            pltpu.make_async_remote_copy(
                src_ref=out_ref.at[:, pl.ds(0, QB), :],
                dst_ref=out_ref.at[:, pl.ds(QB * s, QB), :],
                send_sem=ag_s.at[o - 1],
                recv_sem=ag_r.at[o - 1],
                device_id=(me,),
            ).wait_recv()

        for r in kv_sends + rs_sends + ag_sends:
            r.wait_send()

    return pl.pallas_call(
        body,
        out_shape=jax.ShapeDtypeStruct((B, SQ, F), jnp.float32),
        in_specs=[pl.BlockSpec(memory_space=pltpu.VMEM)] * 4,
        out_specs=pl.BlockSpec(memory_space=pltpu.VMEM),
        scratch_shapes=[
            pltpu.VMEM((N_DEV, 2, B, SKV_LOC, H_LOC, DH), jnp.bfloat16),
            pltpu.VMEM((N_DEV, B, QB, F), jnp.float32),
            pltpu.VMEM((B, SQ, F), jnp.float32),
            pltpu.SemaphoreType.DMA((N_DEV - 1,)),
            pltpu.SemaphoreType.DMA((N_DEV - 1,)),
            pltpu.SemaphoreType.DMA((N_DEV - 1,)),
            pltpu.SemaphoreType.DMA((N_DEV - 1,)),
            pltpu.SemaphoreType.DMA((N_DEV - 1,)),
            pltpu.SemaphoreType.DMA((N_DEV - 1,)),
            pltpu.SemaphoreType.DMA(()),
        ],
    )(x, Wq, kv, Wo)


# baseline (device time: 264622 ns/iter reference)
import jax
import jax.numpy as jnp
from jax import lax
from jax.experimental import pallas as pl
from jax.experimental.pallas import tpu as pltpu

N_DEV = 8
B = 2
SQ = 512
F = 768
H_LOC = 8
DH = 64
SKV_LOC = 512
QB = 64
R = 4


def kernel(x, Wq, K_ext, V_ext, Wo):
    kv = jnp.stack(
        [K_ext.astype(jnp.bfloat16), V_ext.astype(jnp.bfloat16)], axis=0
    ).reshape(2, B, SKV_LOC, 64 * DH)

    def body(x_ref, wq_ref, kv_ref, wo_ref, out_ref,
             kvfull, rsbuf, pscratch,
             kv_s, kv_r, rs_s, rs_r, ag_s, ag_r, loc_sem):
        me = lax.axis_index("i")

        kv_sends = []
        for o in range(1, N_DEV):
            d = (me + o) % N_DEV
            rdma = pltpu.make_async_remote_copy(
                src_ref=kv_ref.at[:, :, :, pl.ds(H_LOC * DH * d, H_LOC * DH)],
                dst_ref=kvfull.at[me],
                send_sem=kv_s.at[o - 1],
                recv_sem=kv_r.at[o - 1],
                device_id=(d,),
            )
            rdma.start()
            kv_sends.append(rdma)

        own = pltpu.make_async_copy(
            kv_ref.at[:, :, :, pl.ds(H_LOC * DH * me, H_LOC * DH)],
            kvfull.at[me],
            loc_sem,
        )
        own.start()

        xv = x_ref[...].reshape(B * SQ, F)
        q2d = lax.dot_general(
            xv, wq_ref[...], (((1,), (0,)), ((), ())),
            preferred_element_type=jnp.float32,
        )
        qb16 = q2d.astype(jnp.bfloat16)

        own.wait()
        for o in range(1, N_DEV):
            s = (me + (N_DEV - o)) % N_DEV
            pltpu.make_async_remote_copy(
                src_ref=kv_ref.at[:, :, :, pl.ds(0, H_LOC * DH)],
                dst_ref=kvfull.at[s],
                send_sem=kv_s.at[o - 1],
                recv_sem=kv_r.at[o - 1],
                device_id=(me,),
            ).wait_recv()

        for b in range(B):
            ctx_halves = []
            for c in range(R):
                kch, vch = [], []
                for s_ in range(N_DEV):
                    for u in range(2):
                        p0 = QB * (R * u + c)
                        kch.append(kvfull[s_, 0, b, p0:p0 + QB, :])
                        vch.append(kvfull[s_, 1, b, p0:p0 + QB, :])
                kc = jnp.concatenate(kch, axis=0)
                vc = jnp.concatenate(vch, axis=0)
                qrows = jnp.concatenate(
                    [
                        qb16[SQ * b + QB * c: SQ * b + QB * (c + 1)],
                        qb16[SQ * b + QB * (c + R): SQ * b + QB * (c + R + 1)],
                    ],
                    axis=0,
                )
                ctx_cols = []
                for h in range(H_LOC):
                    qh = qrows[:, DH * h: DH * (h + 1)]
                    kh = kc[:, DH * h: DH * (h + 1)]
                    vh = vc[:, DH * h: DH * (h + 1)]
                    scores = lax.dot_general(
                        qh, kh, (((1,), (1,)), ((), ())),
                        preferred_element_type=jnp.float32,
                    ) * 0.125
                    m = jnp.max(scores, axis=-1, keepdims=True)
                    w = jnp.exp(scores - m)
                    w = w / jnp.sum(w, axis=-1, keepdims=True)
                    ctx_cols.append(
                        lax.dot_general(
                            w.astype(jnp.bfloat16), vh,
                            (((1,), (0,)), ((), ())),
                            preferred_element_type=jnp.float32,
                        )
                    )
                ctx_halves.append(jnp.concatenate(ctx_cols, axis=1))
            blocks = []
            for qb in range(SQ // QB):
                half = 0 if qb < R else 1
                blocks.append(ctx_halves[qb % R][QB * half: QB * (half + 1)])
            ctx_b = jnp.concatenate(blocks, axis=0)
            pscratch[b] = lax.dot_general(
                ctx_b, wo_ref[...], (((1,), (0,)), ((), ())),
                preferred_element_type=jnp.float32,
            )

        rs_own = pltpu.make_async_copy(
            pscratch.at[:, pl.ds(QB * me, QB), :], rsbuf.at[me], loc_sem
        )
        rs_own.start()
        rs_sends = []
        for o in range(1, N_DEV):
            d = (me + o) % N_DEV
            rdma = pltpu.make_async_remote_copy(
                src_ref=pscratch.at[:, pl.ds(QB * d, QB), :],
                dst_ref=rsbuf.at[me],
                send_sem=rs_s.at[o - 1],
                recv_sem=rs_r.at[o - 1],
                device_id=(d,),
            )
            rdma.start()
            rs_sends.append(rdma)
        rs_own.wait()
        for o in range(1, N_DEV):
            s = (me + (N_DEV - o)) % N_DEV
            pltpu.make_async_remote_copy(
                src_ref=pscratch.at[:, pl.ds(0, QB), :],
                dst_ref=rsbuf.at[s],
                send_sem=rs_s.at[o - 1],
                recv_sem=rs_r.at[o - 1],
                device_id=(me,),
            ).wait_recv()
        red = jnp.sum(rsbuf[...], axis=0)
        out_ref[:, pl.ds(QB * me, QB), :] = red

        ag_sends = []
        for o in range(1, N_DEV):
            d = (me + o) % N_DEV
            rdma = pltpu.make_async_remote_copy(
                src_ref=out_ref.at[:, pl.ds(QB * me, QB), :],
                dst_ref=out_ref.at[:, pl.ds(QB * me, QB), :],
                send_sem=ag_s.at[o - 1],
                recv_sem=ag_r.at[o - 1],
                device_id=(d,),
            )
            rdma.start()
            ag_sends.append(rdma)
        for o in range(1, N_DEV):
            s = (me + (N_DEV - o)) % N_DEV
            pltpu.make_async_remote_copy(
                src_ref=out_ref.at[:, pl.ds(0, QB), :],
                dst_ref=out_ref.at[:, pl.ds(QB * s, QB), :],
                send_sem=ag_s.at[o - 1],
                recv_sem=ag_r.at[o - 1],
                device_id=(me,),
            ).wait_recv()

        for r in kv_sends + rs_sends + ag_sends:
            r.wait_send()

    return pl.pallas_call(
        body,
        out_shape=jax.ShapeDtypeStruct((B, SQ, F), jnp.float32),
        in_specs=[pl.BlockSpec(memory_space=pltpu.VMEM)] * 4,
        out_specs=pl.BlockSpec(memory_space=pltpu.VMEM),
        scratch_shapes=[
            pltpu.VMEM((N_DEV, 2, B, SKV_LOC, H_LOC * DH), jnp.bfloat16),
            pltpu.VMEM((N_DEV, B, QB, F), jnp.float32),
            pltpu.VMEM((B, SQ, F), jnp.float32),
            pltpu.SemaphoreType.DMA((N_DEV - 1,)),
            pltpu.SemaphoreType.DMA((N_DEV - 1,)),
            pltpu.SemaphoreType.DMA((N_DEV - 1,)),
            pltpu.SemaphoreType.DMA((N_DEV - 1,)),
            pltpu.SemaphoreType.DMA((N_DEV - 1,)),
            pltpu.SemaphoreType.DMA((N_DEV - 1,)),
            pltpu.SemaphoreType.DMA(()),
        ],
        compiler_params=pltpu.CompilerParams(
            vmem_limit_bytes=60 * 1024 * 1024,
        ),
    )(x, Wq, kv, Wo)


# device time: 242033 ns/iter; 1.0933x vs baseline; 1.0933x over previous
import jax
import jax.numpy as jnp
from jax import lax
from jax.experimental import pallas as pl
from jax.experimental.pallas import tpu as pltpu

N_DEV = 8
B = 2
SQ = 512
F = 768
H_LOC = 8
DH = 64
SKV_LOC = 512
QB = 64
R = 4
NCH = 4


def kernel(x, Wq, K_ext, V_ext, Wo):
    kv = jnp.stack(
        [K_ext.astype(jnp.bfloat16), V_ext.astype(jnp.bfloat16)], axis=0
    ).reshape(2, B, SKV_LOC, 64 * DH)

    def body(x_ref, wq_ref, kv_ref, wo_ref, out_ref,
             kvfull, pb16, rsbuf, msacc, ctxacc,
             kv_s, kv_r, rs_s, rs_r, ag_s, ag_r, loc_sem):
        me = lax.axis_index("i")

        kv_sends = []
        for o in range(1, N_DEV):
            d = (me + o) % N_DEV
            rdma = pltpu.make_async_remote_copy(
                src_ref=kv_ref.at[:, :, :, pl.ds(H_LOC * DH * d, H_LOC * DH)],
                dst_ref=kvfull.at[o],
                send_sem=kv_s.at[o - 1],
                recv_sem=kv_r.at[o - 1],
                device_id=(d,),
            )
            rdma.start()
            kv_sends.append(rdma)

        own = pltpu.make_async_copy(
            kv_ref.at[:, :, :, pl.ds(H_LOC * DH * me, H_LOC * DH)],
            kvfull.at[0],
            loc_sem.at[0],
        )
        own.start()

        xv = x_ref[...].reshape(B * SQ, F)
        q2d = lax.dot_general(
            xv, wq_ref[...], (((1,), (0,)), ((), ())),
            preferred_element_type=jnp.float32,
        )
        qb16 = q2d.astype(jnp.bfloat16)

        def qrows_of(b, c):
            return jnp.concatenate(
                [
                    qb16[SQ * b + QB * c: SQ * b + QB * (c + 1)],
                    qb16[SQ * b + QB * (c + R): SQ * b + QB * (c + R + 1)],
                ],
                axis=0,
            )

        def attn_chunk(qrows, slots, b, c):
            kch, vch = [], []
            for s_ in slots:
                for u in range(2):
                    p0 = QB * (R * u + c)
                    kch.append(kvfull[s_, 0, b, p0:p0 + QB, :])
                    vch.append(kvfull[s_, 1, b, p0:p0 + QB, :])
            kc = jnp.concatenate(kch, axis=0)
            vc = jnp.concatenate(vch, axis=0)
            m_cols, s_cols, ctx_cols = [], [], []
            for h in range(H_LOC):
                qh = qrows[:, DH * h: DH * (h + 1)]
                kh = kc[:, DH * h: DH * (h + 1)]
                vh = vc[:, DH * h: DH * (h + 1)]
                scores = lax.dot_general(
                    qh, kh, (((1,), (1,)), ((), ())),
                    preferred_element_type=jnp.float32,
                ) * 0.125
                mh = jnp.max(scores, axis=-1, keepdims=True)
                w = jnp.exp(scores - mh)
                sh = jnp.sum(w, axis=-1, keepdims=True)
                ctx_cols.append(
                    lax.dot_general(
                        w.astype(jnp.bfloat16), vh,
                        (((1,), (0,)), ((), ())),
                        preferred_element_type=jnp.float32,
                    )
                )
                m_cols.append(jnp.broadcast_to(mh, (2 * QB, DH)))
                s_cols.append(jnp.broadcast_to(sh, (2 * QB, DH)))
            return (
                jnp.concatenate(m_cols, axis=1),
                jnp.concatenate(s_cols, axis=1),
                jnp.concatenate(ctx_cols, axis=1),
            )

        own.wait()
        for o in range(1, NCH):
            kv_sends[o - 1].wait_recv()
        for b in range(B):
            for c in range(R):
                mA, sA, ctxA = attn_chunk(
                    qrows_of(b, c), range(NCH), b, c)
                msacc[b, c, 0] = mA.astype(jnp.bfloat16)
                msacc[b, c, 1] = sA.astype(jnp.bfloat16)
                ctxacc[b, c] = ctxA.astype(jnp.bfloat16)

        for o in range(NCH, N_DEV):
            kv_sends[o - 1].wait_recv()
        rs_sends = []
        for b in range(B):
            ctx_halves = []
            for c in range(R):
                mB, sB, ctxB = attn_chunk(
                    qrows_of(b, c), range(NCH, N_DEV), b, c)
                mA = msacc[b, c, 0].astype(jnp.float32)
                sA = msacc[b, c, 1].astype(jnp.float32)
                ctxA = ctxacc[b, c].astype(jnp.float32)
                mM = jnp.maximum(mA, mB)
                alpha = jnp.exp(mA - mM)
                beta = jnp.exp(mB - mM)
                denom = sA * alpha + sB * beta
                ctx_halves.append(
                    (ctxA * alpha + ctxB * beta) / denom)
            blocks = []
            for qb in range(SQ // QB):
                half = 0 if qb < R else 1
                blocks.append(ctx_halves[qb % R][QB * half: QB * (half + 1)])
            ctx_b = jnp.concatenate(blocks, axis=0)
            outp = lax.dot_general(
                ctx_b, wo_ref[...], (((1,), (0,)), ((), ())),
                preferred_element_type=jnp.float32,
            )
            pb16[b] = outp.astype(jnp.bfloat16)

            pltpu.make_async_copy(
                pb16.at[b, pl.ds(QB * me, QB), :],
                rsbuf.at[0, b],
                loc_sem.at[1 + b],
            ).start()
            for o in range(1, N_DEV):
                d = (me + o) % N_DEV
                rdma = pltpu.make_async_remote_copy(
                    src_ref=pb16.at[b, pl.ds(QB * d, QB), :],
                    dst_ref=rsbuf.at[o, b],
                    send_sem=rs_s.at[b, o - 1],
                    recv_sem=rs_r.at[b, o - 1],
                    device_id=(d,),
                )
                rdma.start()
                rs_sends.append(rdma)

        for b in range(B):
            pltpu.make_async_copy(
                pb16.at[b, pl.ds(0, QB), :], rsbuf.at[0, b], loc_sem.at[1 + b]
            ).wait()
        for r in rs_sends:
            r.wait_recv()
        red = jnp.sum(rsbuf[...].astype(jnp.float32), axis=0)
        out_ref[:, pl.ds(QB * me, QB), :] = red

        ag_sends = []
        for o in range(1, N_DEV):
            d = (me + o) % N_DEV
            rdma = pltpu.make_async_remote_copy(
                src_ref=out_ref.at[:, pl.ds(QB * me, QB), :],
                dst_ref=out_ref.at[:, pl.ds(QB * me, QB), :],
                send_sem=ag_s.at[o - 1],
                recv_sem=ag_r.at[o - 1],
                device_id=(d,),
            )
            rdma.start()
            ag_sends.append(rdma)
        for o in range(1, N_DEV):
            s = (me + (N_DEV - o)) % N_DEV
            pltpu.make_async_remote_copy(
                src_ref=out_ref.at[:, pl.ds(0, QB), :],
                dst_ref=out_ref.at[:, pl.ds(QB * s, QB), :],
                send_sem=ag_s.at[o - 1],
                recv_sem=ag_r.at[o - 1],
                device_id=(me,),
            ).wait_recv()

        for r in kv_sends + rs_sends + ag_sends:
            r.wait_send()

    return pl.pallas_call(
        body,
        out_shape=jax.ShapeDtypeStruct((B, SQ, F), jnp.float32),
        in_specs=[pl.BlockSpec(memory_space=pltpu.VMEM)] * 4,
        out_specs=pl.BlockSpec(memory_space=pltpu.VMEM),
        scratch_shapes=[
            pltpu.VMEM((N_DEV, 2, B, SKV_LOC, H_LOC * DH), jnp.bfloat16),
            pltpu.VMEM((B, SQ, F), jnp.bfloat16),
            pltpu.VMEM((N_DEV, B, QB, F), jnp.bfloat16),
            pltpu.VMEM((B, R, 2, 2 * QB, H_LOC * DH), jnp.bfloat16),
            pltpu.VMEM((B, R, 2 * QB, H_LOC * DH), jnp.bfloat16),
            pltpu.SemaphoreType.DMA((N_DEV - 1,)),
            pltpu.SemaphoreType.DMA((N_DEV - 1,)),
            pltpu.SemaphoreType.DMA((B, N_DEV - 1)),
            pltpu.SemaphoreType.DMA((B, N_DEV - 1)),
            pltpu.SemaphoreType.DMA((N_DEV - 1,)),
            pltpu.SemaphoreType.DMA((N_DEV - 1,)),
            pltpu.SemaphoreType.DMA((3,)),
        ],
        compiler_params=pltpu.CompilerParams(
            vmem_limit_bytes=62 * 1024 * 1024,
        ),
    )(x, Wq, kv, Wo)
